# speedup vs baseline: 1.5548x; 1.5548x over previous
import jax
import jax.numpy as jnp
from jax.experimental import pallas as pl


def _proj_kernel(ego_ref, wh_ref, y_ref):
    y_ref[...] = jnp.dot(ego_ref[...], wh_ref[...],
                         preferred_element_type=jnp.float32)


def _agg_kernel(y0_ref, ain_ref, aout_ref, y1_ref, y2_ref, out_ref):
    acc = y0_ref[...]
    acc = acc + jnp.dot(ain_ref[...], y1_ref[...],
                        preferred_element_type=jnp.float32)
    acc = acc + jnp.dot(aout_ref[...], y2_ref[...],
                        preferred_element_type=jnp.float32)
    out_ref[...] = jnp.where(acc >= 0, acc, 0.01 * acc)


def kernel(ego_embed, neighbor_in, neighbor_out, W):
    N, D = ego_embed.shape
    OUT = W.shape[1]
    Wh = W.reshape(3, D, OUT).transpose(1, 0, 2).reshape(D, 3 * OUT)

    yall = pl.pallas_call(
        _proj_kernel,
        out_shape=jax.ShapeDtypeStruct((N, 3 * OUT), jnp.float32),
    )(ego_embed, Wh)
    y0 = yall[:, :OUT]
    y1 = yall[:, OUT:2 * OUT]
    y2 = yall[:, 2 * OUT:]

    BM = 200
    out = pl.pallas_call(
        _agg_kernel,
        grid=(N // BM,),
        in_specs=[
            pl.BlockSpec((BM, OUT), lambda i: (i, 0)),
            pl.BlockSpec((BM, N), lambda i: (i, 0)),
            pl.BlockSpec((BM, N), lambda i: (i, 0)),
            pl.BlockSpec((N, OUT), lambda i: (0, 0)),
            pl.BlockSpec((N, OUT), lambda i: (0, 0)),
        ],
        out_specs=pl.BlockSpec((BM, OUT), lambda i: (i, 0)),
        out_shape=jax.ShapeDtypeStruct((N, OUT), jnp.float32),
    )(y0, neighbor_in, neighbor_out, y1, y2)
    return out
